# Initial kernel scaffold; baseline (speedup 1.0000x reference)
import functools

import numpy as np
import jax
import jax.numpy as jnp
from jax.experimental import pallas as pl
from jax.experimental.pallas import tpu as pltpu

_SR = 48000.0
_C = 343.0
_MAX_ORDER = 10
_RIR_LEN = 24000
_TAPS = 81
_HALF = 40
_BETA = 0.9
_SEG = 256
_NQ = 256


@functools.lru_cache(maxsize=None)
def _tables():
    n = np.arange(-_MAX_ORDER, _MAX_ORDER + 1)
    p = np.array([0, 1])
    P, N = np.meshgrid(p, n, indexing="ij")
    sign = (1 - 2 * P).ravel().astype(np.float64)
    off = (2 * N).ravel().astype(np.float64)
    order = (np.abs(N - P) + np.abs(N)).ravel()

    tot = (order[:, None, None] + order[None, :, None] + order[None, None, :])
    keep = tot <= _MAX_ORDER
    ii, jj, kk = np.nonzero(keep)
    K = ii.shape[0]
    Kp = ((K + 127) // 128) * 128

    def pad(v, fill):
        return np.concatenate([v, np.full(Kp - K, fill, np.float64)])

    sx, ox = pad(sign[ii], 0.0), pad(off[ii], 1.0)
    sy, oy = pad(sign[jj], 0.0), pad(off[jj], 1.0)
    sz, oz = pad(sign[kk], 0.0), pad(off[kk], 1.0)
    samp = pad((_BETA ** tot[keep]) / (4.0 * np.pi), 0.0)

    tab = np.zeros((8, Kp), np.float32)
    tab[0], tab[1], tab[2] = sx, ox, sy
    tab[3], tab[4], tab[5] = oy, sz, oz
    tab[6] = samp

    c = np.arange(_SEG, dtype=np.float64)
    pc = np.where(c % 2 == 0, 1.0, -1.0)
    w = np.stack([pc,
                  pc * np.cos(np.pi * c / (_HALF + 1)),
                  pc * np.sin(np.pi * c / (_HALF + 1))])
    wtab = np.repeat(w[:, :, None], 128, axis=2).astype(np.float32)
    return jnp.asarray(tab), jnp.asarray(wtab), Kp


def _rir_kernel(in_ref, tab_ref, w_ref, out_ref, org_ref, *, kp):
    reps = kp // 128
    f0 = in_ref[0, 0, 0] * 10.0
    f1 = in_ref[0, 0, 1] * 10.0
    f2 = in_ref[0, 0, 2] * 10.0
    mic0, mic1, mic2 = in_ref[0, 0, 3] * f0, in_ref[0, 0, 4] * f1, in_ref[0, 0, 5] * f2
    src0, src1, src2 = in_ref[0, 0, 6] * f0, in_ref[0, 0, 7] * f1, in_ref[0, 0, 8] * f2

    sx = tab_ref[0:1, :]
    ox = tab_ref[1:2, :]
    sy = tab_ref[2:3, :]
    oy = tab_ref[3:4, :]
    sz = tab_ref[4:5, :]
    oz = tab_ref[5:6, :]
    samp = tab_ref[6:7, :]

    dx = sx * src0 + ox * f0 - mic0
    dy = sy * src1 + oy * f1 - mic1
    dz = sz * src2 + oz * f2 - mic2
    dist = jnp.sqrt(dx * dx + dy * dy + dz * dz)
    amp = samp / dist
    tau = dist * np.float32(_SR) / np.float32(_C)
    i0 = jnp.floor(tau)
    frac = tau - i0
    base = i0 + np.float32(_HALF)
    qf = jnp.floor(base * np.float32(1.0 / 128.0))
    rf = base - 128.0 * qf
    rrf = rf + np.float32(_HALF) + frac
    u = rrf * np.float32(np.pi / (_HALF + 1))
    cu = jnp.cos(u)
    su = jnp.sin(u)
    sfrac = jnp.sin(np.float32(np.pi) * frac)
    par = 1.0 - 2.0 * (rf - 2.0 * jnp.floor(rf * 0.5))
    s_img = np.float32(-0.5 / np.pi) * amp * par * sfrac

    ci = jax.lax.broadcasted_iota(jnp.float32, (_SEG, kp), 0)
    tt = ci - rrf
    w1 = pltpu.repeat(w_ref[0], reps, axis=1)
    w2 = pltpu.repeat(w_ref[1], reps, axis=1)
    w3 = pltpu.repeat(w_ref[2], reps, axis=1)
    win = w1 + w2 * cu + w3 * su
    val = (s_img / tt) * win
    val = jnp.where(jnp.abs(tt) <= np.float32(_HALF), val, 0.0)
    val = jnp.where(tt == 0.0, amp, val)

    oh = jnp.where(ci == qf, 1.0, 0.0)
    m2 = jax.lax.dot_general(
        oh, val, (((1,), (1,)), ((), ())),
        preferred_element_type=jnp.float32,
        precision=jax.lax.Precision.HIGHEST)

    out_ref[0, 0:1, :] = m2[0:1, 0:128]
    out_ref[0, 1:_NQ, :] = m2[1:_NQ, 0:128] + m2[0:_NQ - 1, 128:_SEG]

    dd0, dd1, dd2 = mic0 - src0, mic1 - src1, mic2 - src2
    dd = jnp.sqrt(dd0 * dd0 + dd1 * dd1 + dd2 * dd2)
    org_ref[0, 0, :] = jnp.full((128,), 40.0 + dd * np.float32(_SR) / np.float32(_C),
                                dtype=jnp.float32)


def kernel(input):
    tab, wtab, kp = _tables()
    B = input.shape[0]
    in3 = input.reshape(B, 1, 9)

    out3, org3 = pl.pallas_call(
        functools.partial(_rir_kernel, kp=kp),
        grid=(B,),
        in_specs=[
            pl.BlockSpec((1, 1, 9), lambda b: (b, 0, 0)),
            pl.BlockSpec((8, kp), lambda b: (0, 0)),
            pl.BlockSpec((3, _SEG, 128), lambda b: (0, 0, 0)),
        ],
        out_specs=[
            pl.BlockSpec((1, _NQ, 128), lambda b: (b, 0, 0)),
            pl.BlockSpec((1, 1, 128), lambda b: (b, 0, 0)),
        ],
        out_shape=[
            jax.ShapeDtypeStruct((B, _NQ, 128), jnp.float32),
            jax.ShapeDtypeStruct((B, 1, 128), jnp.float32),
        ],
        compiler_params=pltpu.CompilerParams(
            dimension_semantics=("parallel",),
            vmem_limit_bytes=48 * 1024 * 1024,
        ),
        name="shoebox_rir",
    )(in3, tab, wtab)

    rir = out3.reshape(B, _NQ * 128)[:, :_RIR_LEN]
    origin = org3[:, 0, 0]
    return rir, origin


# speedup vs baseline: 3497.3599x; 3497.3599x over previous
import functools

import numpy as np
import jax
import jax.numpy as jnp
from jax.experimental import pallas as pl
from jax.experimental.pallas import tpu as pltpu

_SR = 48000.0
_C = 343.0
_MAX_ORDER = 10
_RIR_LEN = 24000
_TAPS = 81
_HALF = 40
_BETA = 0.9
_SEG = 256
_NQ = 256


@functools.lru_cache(maxsize=None)
def _tables():
    n = np.arange(-_MAX_ORDER, _MAX_ORDER + 1)
    p = np.array([0, 1])
    P, N = np.meshgrid(p, n, indexing="ij")
    sign = (1 - 2 * P).ravel().astype(np.float64)
    off = (2 * N).ravel().astype(np.float64)
    order = (np.abs(N - P) + np.abs(N)).ravel()

    tot = (order[:, None, None] + order[None, :, None] + order[None, None, :])
    keep = tot <= _MAX_ORDER
    ii, jj, kk = np.nonzero(keep)
    K = ii.shape[0]
    Kp = ((K + 127) // 128) * 128

    def pad(v, fill):
        return np.concatenate([v, np.full(Kp - K, fill, np.float64)])

    sx, ox = pad(sign[ii], 0.0), pad(off[ii], 1.0)
    sy, oy = pad(sign[jj], 0.0), pad(off[jj], 1.0)
    sz, oz = pad(sign[kk], 0.0), pad(off[kk], 1.0)
    samp = pad((_BETA ** tot[keep]) / (4.0 * np.pi), 0.0)

    tab = np.zeros((8, Kp), np.float32)
    tab[0], tab[1], tab[2] = sx, ox, sy
    tab[3], tab[4], tab[5] = oy, sz, oz
    tab[6] = samp

    c = np.arange(_SEG, dtype=np.float64)
    pc = np.where(c % 2 == 0, 1.0, -1.0)
    w = np.stack([pc,
                  pc * np.cos(np.pi * c / (_HALF + 1)),
                  pc * np.sin(np.pi * c / (_HALF + 1))])
    wtab = np.repeat(w[:, :, None], 128, axis=2).astype(np.float32)
    return jnp.asarray(tab), jnp.asarray(wtab), Kp


def _rir_kernel(in_ref, tab_ref, w_ref, out_ref, org_ref, *, kp):
    reps = kp // 128
    f0 = in_ref[0, 0, 0] * 10.0
    f1 = in_ref[0, 0, 1] * 10.0
    f2 = in_ref[0, 0, 2] * 10.0
    mic0, mic1, mic2 = in_ref[0, 0, 3] * f0, in_ref[0, 0, 4] * f1, in_ref[0, 0, 5] * f2
    src0, src1, src2 = in_ref[0, 0, 6] * f0, in_ref[0, 0, 7] * f1, in_ref[0, 0, 8] * f2

    sx = tab_ref[0:1, :]
    ox = tab_ref[1:2, :]
    sy = tab_ref[2:3, :]
    oy = tab_ref[3:4, :]
    sz = tab_ref[4:5, :]
    oz = tab_ref[5:6, :]
    samp = tab_ref[6:7, :]

    dx = sx * src0 + ox * f0 - mic0
    dy = sy * src1 + oy * f1 - mic1
    dz = sz * src2 + oz * f2 - mic2
    dist = jnp.sqrt(dx * dx + dy * dy + dz * dz)
    amp = samp / dist
    tau = dist * np.float32(_SR) / np.float32(_C)
    i0 = jnp.floor(tau)
    frac = tau - i0
    base = i0 + np.float32(_HALF)
    qf = jnp.floor(base * np.float32(1.0 / 128.0))
    rf = base - 128.0 * qf
    rrf = rf + np.float32(_HALF) + frac
    u = rrf * np.float32(np.pi / (_HALF + 1))
    cu = jnp.cos(u)
    su = jnp.sin(u)
    sfrac = jnp.sin(np.float32(np.pi) * frac)
    par = 1.0 - 2.0 * (rf - 2.0 * jnp.floor(rf * 0.5))
    s_img = np.float32(-0.5 / np.pi) * amp * par * sfrac

    ci_i = jax.lax.broadcasted_iota(jnp.int32, (_SEG, kp), 0)
    ci = ci_i.astype(jnp.float32)
    tt = ci - rrf
    w1 = pltpu.repeat(w_ref[0], reps, axis=1)
    w2 = pltpu.repeat(w_ref[1], reps, axis=1)
    w3 = pltpu.repeat(w_ref[2], reps, axis=1)
    win = w1 + w2 * cu + w3 * su
    val = (s_img / tt) * win
    val = jnp.where(jnp.abs(tt) <= np.float32(_HALF), val, 0.0)
    val = jnp.where(tt == 0.0, amp, val)

    oh = jnp.where(ci_i == qf.astype(jnp.int32), 1.0, 0.0)
    m2 = jax.lax.dot_general(
        oh, val, (((1,), (1,)), ((), ())),
        preferred_element_type=jnp.float32,
        precision=jax.lax.Precision.HIGHEST)

    out_ref[0, 0:1, :] = m2[0:1, 0:128]
    out_ref[0, 1:_NQ, :] = m2[1:_NQ, 0:128] + m2[0:_NQ - 1, 128:_SEG]

    dd0, dd1, dd2 = mic0 - src0, mic1 - src1, mic2 - src2
    dd = jnp.sqrt(dd0 * dd0 + dd1 * dd1 + dd2 * dd2)
    org_ref[0, 0, :] = jnp.full((128,), 40.0 + dd * np.float32(_SR) / np.float32(_C),
                                dtype=jnp.float32)


def kernel(input):
    tab, wtab, kp = _tables()
    B = input.shape[0]
    in3 = input.reshape(B, 1, 9)

    out3, org3 = pl.pallas_call(
        functools.partial(_rir_kernel, kp=kp),
        grid=(B,),
        in_specs=[
            pl.BlockSpec((1, 1, 9), lambda b: (b, 0, 0)),
            pl.BlockSpec((8, kp), lambda b: (0, 0)),
            pl.BlockSpec((3, _SEG, 128), lambda b: (0, 0, 0)),
        ],
        out_specs=[
            pl.BlockSpec((1, _NQ, 128), lambda b: (b, 0, 0)),
            pl.BlockSpec((1, 1, 128), lambda b: (b, 0, 0)),
        ],
        out_shape=[
            jax.ShapeDtypeStruct((B, _NQ, 128), jnp.float32),
            jax.ShapeDtypeStruct((B, 1, 128), jnp.float32),
        ],
        compiler_params=pltpu.CompilerParams(
            dimension_semantics=("parallel",),
            vmem_limit_bytes=48 * 1024 * 1024,
        ),
        name="shoebox_rir",
    )(in3, tab, wtab)

    rir = out3.reshape(B, _NQ * 128)[:, :_RIR_LEN]
    origin = org3[:, 0, 0]
    return rir, origin


# speedup vs baseline: 5713.4214x; 1.6336x over previous
import functools

import numpy as np
import jax
import jax.numpy as jnp
from jax.experimental import pallas as pl
from jax.experimental.pallas import tpu as pltpu

_SR = 48000.0
_C = 343.0
_MAX_ORDER = 10
_RIR_LEN = 24000
_TAPS = 81
_HALF = 40
_BETA = 0.9
_SEG = 256
_NQ = 256


@functools.lru_cache(maxsize=None)
def _tables():
    n = np.arange(-_MAX_ORDER, _MAX_ORDER + 1)
    p = np.array([0, 1])
    P, N = np.meshgrid(p, n, indexing="ij")
    sign = (1 - 2 * P).ravel().astype(np.float64)
    off = (2 * N).ravel().astype(np.float64)
    order = (np.abs(N - P) + np.abs(N)).ravel()

    tot = (order[:, None, None] + order[None, :, None] + order[None, None, :])
    keep = tot <= _MAX_ORDER
    ii, jj, kk = np.nonzero(keep)
    K = ii.shape[0]
    Kp = ((K + 127) // 128) * 128

    def pad(v, fill):
        return np.concatenate([v, np.full(Kp - K, fill, np.float64)])

    sx, ox = pad(sign[ii], 0.0), pad(off[ii], 1.0)
    sy, oy = pad(sign[jj], 0.0), pad(off[jj], 1.0)
    sz, oz = pad(sign[kk], 0.0), pad(off[kk], 1.0)
    samp = pad((_BETA ** tot[keep]) / (4.0 * np.pi), 0.0)

    tab = np.zeros((8, Kp), np.float32)
    tab[0], tab[1], tab[2] = sx, ox, sy
    tab[3], tab[4], tab[5] = oy, sz, oz
    tab[6] = samp

    c = np.arange(_SEG, dtype=np.float64)
    pc = np.where(c % 2 == 0, 1.0, -1.0)
    w = np.stack([pc,
                  pc * np.cos(np.pi * c / (_HALF + 1)),
                  pc * np.sin(np.pi * c / (_HALF + 1))])
    wtab = np.repeat(w[:, :, None], 128, axis=2).astype(np.float32)
    return jnp.asarray(tab), jnp.asarray(wtab), Kp


def _rir_kernel(in_ref, tab_ref, w_ref, out_ref, org_ref, *, kp):
    reps = kp // 128
    f0 = in_ref[0, 0, 0] * 10.0
    f1 = in_ref[0, 0, 1] * 10.0
    f2 = in_ref[0, 0, 2] * 10.0
    mic0, mic1, mic2 = in_ref[0, 0, 3] * f0, in_ref[0, 0, 4] * f1, in_ref[0, 0, 5] * f2
    src0, src1, src2 = in_ref[0, 0, 6] * f0, in_ref[0, 0, 7] * f1, in_ref[0, 0, 8] * f2

    sx = tab_ref[0:1, :]
    ox = tab_ref[1:2, :]
    sy = tab_ref[2:3, :]
    oy = tab_ref[3:4, :]
    sz = tab_ref[4:5, :]
    oz = tab_ref[5:6, :]
    samp = tab_ref[6:7, :]

    dx = sx * src0 + ox * f0 - mic0
    dy = sy * src1 + oy * f1 - mic1
    dz = sz * src2 + oz * f2 - mic2
    dist = jnp.sqrt(dx * dx + dy * dy + dz * dz)
    amp = samp / dist
    tau = dist * np.float32(_SR) / np.float32(_C)
    i0 = jnp.floor(tau)
    frac = tau - i0
    base = i0 + np.float32(_HALF)
    qf = jnp.floor(base * np.float32(1.0 / 128.0))
    rf = base - 128.0 * qf
    rrf = rf + np.float32(_HALF) + frac
    u = rrf * np.float32(np.pi / (_HALF + 1))
    cu = jnp.cos(u)
    su = jnp.sin(u)
    sfrac = jnp.sin(np.float32(np.pi) * frac)
    par = 1.0 - 2.0 * (rf - 2.0 * jnp.floor(rf * 0.5))
    s_img = np.float32(-0.5 / np.pi) * amp * par * sfrac

    ci_i = jax.lax.broadcasted_iota(jnp.int32, (_SEG, kp), 0)
    ci = ci_i.astype(jnp.float32)
    tt = ci - rrf
    w1 = pltpu.repeat(w_ref[0], reps, axis=1)
    w2 = pltpu.repeat(w_ref[1], reps, axis=1)
    w3 = pltpu.repeat(w_ref[2], reps, axis=1)
    win = w1 + w2 * cu + w3 * su
    val = (s_img / tt) * win
    val = jnp.where(jnp.abs(tt) <= np.float32(_HALF), val, 0.0)
    val = jnp.where(tt == 0.0, amp, val)

    oh = jnp.where(ci_i == qf.astype(jnp.int32), 1.0, 0.0)
    m2 = jax.lax.dot_general(
        oh, val, (((1,), (1,)), ((), ())),
        preferred_element_type=jnp.float32,
        precision=jax.lax.Precision.DEFAULT)

    out_ref[0, 0:1, :] = m2[0:1, 0:128]
    out_ref[0, 1:_NQ, :] = m2[1:_NQ, 0:128] + m2[0:_NQ - 1, 128:_SEG]

    dd0, dd1, dd2 = mic0 - src0, mic1 - src1, mic2 - src2
    dd = jnp.sqrt(dd0 * dd0 + dd1 * dd1 + dd2 * dd2)
    org_ref[0, 0, :] = jnp.full((128,), 40.0 + dd * np.float32(_SR) / np.float32(_C),
                                dtype=jnp.float32)


def kernel(input):
    tab, wtab, kp = _tables()
    B = input.shape[0]
    in3 = input.reshape(B, 1, 9)

    out3, org3 = pl.pallas_call(
        functools.partial(_rir_kernel, kp=kp),
        grid=(B,),
        in_specs=[
            pl.BlockSpec((1, 1, 9), lambda b: (b, 0, 0)),
            pl.BlockSpec((8, kp), lambda b: (0, 0)),
            pl.BlockSpec((3, _SEG, 128), lambda b: (0, 0, 0)),
        ],
        out_specs=[
            pl.BlockSpec((1, _NQ, 128), lambda b: (b, 0, 0)),
            pl.BlockSpec((1, 1, 128), lambda b: (b, 0, 0)),
        ],
        out_shape=[
            jax.ShapeDtypeStruct((B, _NQ, 128), jnp.float32),
            jax.ShapeDtypeStruct((B, 1, 128), jnp.float32),
        ],
        compiler_params=pltpu.CompilerParams(
            dimension_semantics=("parallel",),
            vmem_limit_bytes=48 * 1024 * 1024,
        ),
        name="shoebox_rir",
    )(in3, tab, wtab)

    rir = out3.reshape(B, _NQ * 128)[:, :_RIR_LEN]
    origin = org3[:, 0, 0]
    return rir, origin


# speedup vs baseline: 6668.8354x; 1.1672x over previous
import functools

import numpy as np
import jax
import jax.numpy as jnp
from jax.experimental import pallas as pl
from jax.experimental.pallas import tpu as pltpu

_SR = 48000.0
_C = 343.0
_MAX_ORDER = 10
_RIR_LEN = 24000
_TAPS = 81
_HALF = 40
_BETA = 0.9
_SEG = 256
_NQ = 256


@functools.lru_cache(maxsize=None)
def _tables():
    n = np.arange(-_MAX_ORDER, _MAX_ORDER + 1)
    p = np.array([0, 1])
    P, N = np.meshgrid(p, n, indexing="ij")
    sign = (1 - 2 * P).ravel().astype(np.float64)
    off = (2 * N).ravel().astype(np.float64)
    order = (np.abs(N - P) + np.abs(N)).ravel()

    tot = (order[:, None, None] + order[None, :, None] + order[None, None, :])
    keep = tot <= _MAX_ORDER
    ii, jj, kk = np.nonzero(keep)
    K = ii.shape[0]
    Kp = ((K + 127) // 128) * 128

    def pad(v, fill):
        return np.concatenate([v, np.full(Kp - K, fill, np.float64)])

    sx, ox = pad(sign[ii], 0.0), pad(off[ii], 1.0)
    sy, oy = pad(sign[jj], 0.0), pad(off[jj], 1.0)
    sz, oz = pad(sign[kk], 0.0), pad(off[kk], 1.0)
    samp = pad((_BETA ** tot[keep]) / (4.0 * np.pi), 0.0)

    tab = np.zeros((8, Kp), np.float32)
    tab[0], tab[1], tab[2] = sx, ox, sy
    tab[3], tab[4], tab[5] = oy, sz, oz
    tab[6] = samp

    c = np.arange(_SEG, dtype=np.float64)
    pc = np.where(c % 2 == 0, 1.0, -1.0)
    w = np.stack([pc,
                  pc * np.cos(np.pi * c / (_HALF + 1)),
                  pc * np.sin(np.pi * c / (_HALF + 1))])
    wtab = np.repeat(w[:, :, None], 128, axis=2).astype(np.float32)
    return jnp.asarray(tab), jnp.asarray(wtab), Kp


def _rir_kernel(in_ref, tab_ref, w_ref, out_ref, org_ref, *, kp, nb):
    reps = kp // 128
    room = in_ref[:, 0:3] * 10.0
    mic = in_ref[:, 3:6] * room
    src = in_ref[:, 6:9] * room

    sx = tab_ref[0:1, :]
    ox = tab_ref[1:2, :]
    sy = tab_ref[2:3, :]
    oy = tab_ref[3:4, :]
    sz = tab_ref[4:5, :]
    oz = tab_ref[5:6, :]
    samp = tab_ref[6:7, :]

    dx = sx * src[:, 0:1] + ox * room[:, 0:1] - mic[:, 0:1]
    dy = sy * src[:, 1:2] + oy * room[:, 1:2] - mic[:, 1:2]
    dz = sz * src[:, 2:3] + oz * room[:, 2:3] - mic[:, 2:3]
    dist = jnp.sqrt(dx * dx + dy * dy + dz * dz)
    amp = samp / dist
    tau = dist * np.float32(_SR) / np.float32(_C)
    i0 = jnp.floor(tau)
    frac = tau - i0
    base = i0 + np.float32(_HALF)
    qf = jnp.floor(base * np.float32(1.0 / 128.0))
    rf = base - 128.0 * qf
    rrf = rf + np.float32(_HALF) + frac
    u = rrf * np.float32(np.pi / (_HALF + 1))
    cu = jnp.cos(u)
    su = jnp.sin(u)
    sfrac = jnp.sin(np.float32(np.pi) * frac)
    par = 1.0 - 2.0 * (rf - 2.0 * jnp.floor(rf * 0.5))
    s_img = np.float32(-0.5 / np.pi) * amp * par * sfrac

    ci_i = jax.lax.broadcasted_iota(jnp.int32, (_SEG, kp), 0)
    ci = ci_i.astype(jnp.float32)
    w1 = pltpu.repeat(w_ref[0], reps, axis=1)
    w2 = pltpu.repeat(w_ref[1], reps, axis=1)
    w3 = pltpu.repeat(w_ref[2], reps, axis=1)

    for b in range(nb):
        bs = slice(b, b + 1)
        tt = ci - rrf[bs, :]
        win = w1 + w2 * cu[bs, :] + w3 * su[bs, :]
        val = (s_img[bs, :] / tt) * win
        val = jnp.where(jnp.abs(tt) <= np.float32(_HALF), val, 0.0)
        val = jnp.where(tt == 0.0, amp[bs, :], val)
        oh = jnp.where(ci == qf[bs, :], 1.0, 0.0)
        m2 = jax.lax.dot_general(
            oh, val, (((1,), (1,)), ((), ())),
            preferred_element_type=jnp.float32,
            precision=jax.lax.Precision.DEFAULT)
        out_ref[b, 0:1, :] = m2[0:1, 0:128]
        out_ref[b, 1:_NQ, :] = m2[1:_NQ, 0:128] + m2[0:_NQ - 1, 128:_SEG]

    dd = mic - src
    nrm = jnp.sqrt(jnp.sum(dd * dd, axis=1, keepdims=True))
    org_ref[:, :] = jnp.broadcast_to(
        40.0 + nrm * np.float32(_SR) / np.float32(_C), org_ref.shape)


def kernel(input):
    tab, wtab, kp = _tables()
    B = input.shape[0]

    out3, org2 = pl.pallas_call(
        functools.partial(_rir_kernel, kp=kp, nb=B),
        out_shape=[
            jax.ShapeDtypeStruct((B, _NQ, 128), jnp.float32),
            jax.ShapeDtypeStruct((B, 128), jnp.float32),
        ],
        compiler_params=pltpu.CompilerParams(
            vmem_limit_bytes=48 * 1024 * 1024,
        ),
        name="shoebox_rir",
    )(input, tab, wtab)

    rir = out3.reshape(B, _NQ * 128)[:, :_RIR_LEN]
    origin = org2[:, 0]
    return rir, origin


# speedup vs baseline: 7521.5504x; 1.1279x over previous
import functools

import numpy as np
import jax
import jax.numpy as jnp
from jax.experimental import pallas as pl
from jax.experimental.pallas import tpu as pltpu

_SR = 48000.0
_C = 343.0
_MAX_ORDER = 10
_RIR_LEN = 24000
_TAPS = 81
_HALF = 40
_BETA = 0.9
_SEG = 216
_NQ = 208


@functools.lru_cache(maxsize=None)
def _tables():
    n = np.arange(-_MAX_ORDER, _MAX_ORDER + 1)
    p = np.array([0, 1])
    P, N = np.meshgrid(p, n, indexing="ij")
    sign = (1 - 2 * P).ravel().astype(np.float64)
    off = (2 * N).ravel().astype(np.float64)
    order = (np.abs(N - P) + np.abs(N)).ravel()

    tot = (order[:, None, None] + order[None, :, None] + order[None, None, :])
    keep = tot <= _MAX_ORDER
    ii, jj, kk = np.nonzero(keep)
    K = ii.shape[0]
    Kp = ((K + 127) // 128) * 128

    def pad(v, fill):
        return np.concatenate([v, np.full(Kp - K, fill, np.float64)])

    sx, ox = pad(sign[ii], 0.0), pad(off[ii], 1.0)
    sy, oy = pad(sign[jj], 0.0), pad(off[jj], 1.0)
    sz, oz = pad(sign[kk], 0.0), pad(off[kk], 1.0)
    samp = pad((_BETA ** tot[keep]) / (4.0 * np.pi), 0.0)

    tab = np.zeros((8, Kp), np.float32)
    tab[0], tab[1], tab[2] = sx, ox, sy
    tab[3], tab[4], tab[5] = oy, sz, oz
    tab[6] = samp

    c = np.arange(_SEG, dtype=np.float64)
    pc = np.where(c % 2 == 0, 1.0, -1.0)
    w = np.stack([pc,
                  pc * np.cos(np.pi * c / (_HALF + 1)),
                  pc * np.sin(np.pi * c / (_HALF + 1))])
    wtab = np.repeat(w[:, :, None], 128, axis=2).astype(np.float32)
    return jnp.asarray(tab), jnp.asarray(wtab), Kp


def _rir_kernel(in_ref, tab_ref, w_ref, out_ref, org_ref, *, kp, nb):
    reps = kp // 128
    room = in_ref[:, 0:3] * 10.0
    mic = in_ref[:, 3:6] * room
    src = in_ref[:, 6:9] * room

    sx = tab_ref[0:1, :]
    ox = tab_ref[1:2, :]
    sy = tab_ref[2:3, :]
    oy = tab_ref[3:4, :]
    sz = tab_ref[4:5, :]
    oz = tab_ref[5:6, :]
    samp = tab_ref[6:7, :]

    dx = sx * src[:, 0:1] + ox * room[:, 0:1] - mic[:, 0:1]
    dy = sy * src[:, 1:2] + oy * room[:, 1:2] - mic[:, 1:2]
    dz = sz * src[:, 2:3] + oz * room[:, 2:3] - mic[:, 2:3]
    dist = jnp.sqrt(dx * dx + dy * dy + dz * dz)
    amp = samp / dist
    tau = dist * np.float32(_SR) / np.float32(_C)
    i0 = jnp.floor(tau)
    frac = tau - i0
    base = i0 + np.float32(_HALF)
    qf = jnp.floor(base * np.float32(1.0 / 128.0))
    rf = base - 128.0 * qf
    rrf = rf + np.float32(_HALF) + frac
    u = rrf * np.float32(np.pi / (_HALF + 1))
    cu = jnp.cos(u)
    su = jnp.sin(u)
    sfrac = jnp.sin(np.float32(np.pi) * frac)
    par = 1.0 - 2.0 * (rf - 2.0 * jnp.floor(rf * 0.5))
    s_img = np.float32(-0.5 / np.pi) * amp * par * sfrac

    ci_i = jax.lax.broadcasted_iota(jnp.int32, (_SEG, kp), 0)
    ci = ci_i.astype(jnp.float32)
    w1 = pltpu.repeat(w_ref[0], reps, axis=1)
    w2 = pltpu.repeat(w_ref[1], reps, axis=1)
    w3 = pltpu.repeat(w_ref[2], reps, axis=1)

    ci_oh = ci[0:_NQ, :]
    for b in range(nb):
        bs = slice(b, b + 1)
        tt = ci - rrf[bs, :]
        win = w1 + w2 * cu[bs, :] + w3 * su[bs, :]
        val = (s_img[bs, :] / tt) * win
        val = jnp.where(jnp.abs(tt) <= np.float32(_HALF), val, 0.0)
        val = jnp.where(tt == 0.0, amp[bs, :], val)
        oh = jnp.where(ci_oh == qf[bs, :], 1.0, 0.0)
        m2 = jax.lax.dot_general(
            oh, val, (((1,), (1,)), ((), ())),
            preferred_element_type=jnp.float32,
            precision=jax.lax.Precision.DEFAULT)
        tailp = jnp.concatenate(
            [m2[:, 128:_SEG],
             jnp.zeros((_NQ, 128 - (_SEG - 128)), jnp.float32)], axis=1)
        out_ref[b, 0:1, :] = m2[0:1, 0:128]
        out_ref[b, 1:_NQ, :] = m2[1:_NQ, 0:128] + tailp[0:_NQ - 1, :]

    dd = mic - src
    nrm = jnp.sqrt(jnp.sum(dd * dd, axis=1, keepdims=True))
    org_ref[:, :] = jnp.broadcast_to(
        40.0 + nrm * np.float32(_SR) / np.float32(_C), org_ref.shape)


def kernel(input):
    tab, wtab, kp = _tables()
    B = input.shape[0]

    out3, org2 = pl.pallas_call(
        functools.partial(_rir_kernel, kp=kp, nb=B),
        out_shape=[
            jax.ShapeDtypeStruct((B, _NQ, 128), jnp.float32),
            jax.ShapeDtypeStruct((B, 128), jnp.float32),
        ],
        compiler_params=pltpu.CompilerParams(
            vmem_limit_bytes=48 * 1024 * 1024,
        ),
        name="shoebox_rir",
    )(input, tab, wtab)

    rir = out3.reshape(B, _NQ * 128)[:, :_RIR_LEN]
    origin = org2[:, 0]
    return rir, origin
